# Initial kernel scaffold; baseline (speedup 1.0000x reference)
import jax
import jax.numpy as jnp
from jax.experimental import pallas as pl


def kernel(i_goods_id, i_shop_id, i_cate_id, visited_goods_ids, visited_shop_ids, visited_cate_ids, emb_table, W1, b1, g1, be1, a1, W2, b2, g2, be2, a2, W3, b3):
    raise NotImplementedError("write your pallas kernel here")


# speedup vs baseline: 1.2920x; 1.2920x over previous
import functools

import jax
import jax.numpy as jnp
from jax import lax
from jax.experimental import pallas as pl
from jax.experimental.pallas import tpu as pltpu
from jax.experimental.pallas import tpu_sc as plsc

B, L, E, V = 4096, 200, 16, 1000000
H1, H2, OUT = 200, 80, 2

NW = 32
GROW = 128

IT_G = 3
IT_PER_W = IT_G * GROW
SE_G = 12
SE_CHUNK = SE_G * GROW
SE_NCH = 50
SE_PER_W = SE_NCH * SE_CHUNK


def _gather_tec(item_idx_hbm, series_idx_hbm, table_hbm,
                out_item, out_series, idx_v, rows_v, sem):
    wid = lax.axis_index("s") * 2 + lax.axis_index("c")

    pltpu.sync_copy(item_idx_hbm.at[wid], idx_v.at[pl.ds(0, IT_G)])
    descs = [
        pltpu.async_copy(table_hbm.at[idx_v.at[j]],
                         rows_v.at[pl.ds(j * GROW, GROW)], sem)
        for j in range(IT_G)
    ]
    for d in descs:
        d.wait()
    pltpu.sync_copy(rows_v.at[pl.ds(0, IT_PER_W)],
                    out_item.at[pl.ds(wid * IT_PER_W, IT_PER_W)])

    def body(c, carry):
        pltpu.sync_copy(series_idx_hbm.at[wid, c], idx_v)
        ds = [
            pltpu.async_copy(table_hbm.at[idx_v.at[j]],
                             rows_v.at[pl.ds(j * GROW, GROW)], sem)
            for j in range(SE_G)
        ]
        for d in ds:
            d.wait()
        off = wid * SE_PER_W + c * SE_CHUNK
        pltpu.sync_copy(rows_v.at[pl.ds(0, SE_CHUNK)],
                        out_series.at[pl.ds(off, SE_CHUNK)])
        return carry

    lax.fori_loop(0, SE_NCH, body, 0)


def _sc_gather(item_idx, series_idx, table):
    mesh = plsc.VectorSubcoreMesh(core_axis_name="c", subcore_axis_name="s")
    f = functools.partial(
        pl.kernel, mesh=mesh,
        compiler_params=pltpu.CompilerParams(use_tc_tiling_on_sc=False),
        out_type=(
            jax.ShapeDtypeStruct((NW * IT_PER_W, E), jnp.float32),
            jax.ShapeDtypeStruct((NW * SE_PER_W, E), jnp.float32),
        ),
        scratch_types=[
            pltpu.VMEM((SE_G, GROW), jnp.int32),
            pltpu.VMEM((SE_CHUNK, E), jnp.float32),
            pltpu.SemaphoreType.DMA,
        ],
    )(_gather_tec)
    return f(item_idx, series_idx, table)


def _attn_body(mf_ref, xi_ref, xs_ref, pooled_ref):
    xs = xs_ref[...]
    xi = xi_ref[...]
    scores = jnp.sum(xs * xi[:, None, :], axis=2)
    ms = scores * mf_ref[...]
    pooled_ref[...] = jnp.sum(xs * ms[:, :, None], axis=1)


def _attention(maskf, x_item, x_series, bb=128):
    grid = (B // bb,)
    return pl.pallas_call(
        _attn_body,
        grid=grid,
        in_specs=[
            pl.BlockSpec((bb, L), lambda i: (i, 0)),
            pl.BlockSpec((bb, 3 * E), lambda i: (i, 0)),
            pl.BlockSpec((bb, L, 3 * E), lambda i: (i, 0, 0)),
        ],
        out_specs=pl.BlockSpec((bb, 3 * E), lambda i: (i, 0)),
        out_shape=jax.ShapeDtypeStruct((B, 3 * E), jnp.float32),
    )(maskf, x_item, x_series)


def _sigmoid(x):
    return 1.0 / (1.0 + jnp.exp(-x))


def _ln(x, gamma, beta, eps=1e-3):
    mu = jnp.mean(x, axis=-1, keepdims=True)
    var = jnp.mean((x - mu) ** 2, axis=-1, keepdims=True)
    return gamma * (x - mu) / jnp.sqrt(var + eps) + beta


def _dice_act(x, alpha, eps=1e-3):
    mu = jnp.mean(x, axis=0, keepdims=True)
    var = jnp.mean((x - mu) ** 2, axis=0, keepdims=True)
    xn = (x - mu) / jnp.sqrt(var + eps)
    p = _sigmoid(xn)
    return alpha * (1.0 - p) * x + p * x


def _mlp_body(xi_ref, pooled_ref, w1_ref, b1_ref, g1_ref, be1_ref, a1_ref,
              w2_ref, b2_ref, g2_ref, be2_ref, a2_ref, w3_ref, b3_ref, out_ref):
    x = jnp.concatenate([xi_ref[...], pooled_ref[...]], axis=1)
    h = jnp.dot(x, w1_ref[...], preferred_element_type=jnp.float32) + b1_ref[...]
    h = _ln(h, g1_ref[...], be1_ref[...])
    h = _dice_act(h, a1_ref[...])
    h = jnp.dot(h, w2_ref[...], preferred_element_type=jnp.float32) + b2_ref[...]
    h = _ln(h, g2_ref[...], be2_ref[...])
    h = _dice_act(h, a2_ref[...])
    logits = jnp.dot(h, w3_ref[...], preferred_element_type=jnp.float32) + b3_ref[...]
    m = jnp.max(logits, axis=-1, keepdims=True)
    e = jnp.exp(logits - m)
    out_ref[...] = e / jnp.sum(e, axis=-1, keepdims=True)


def _mlp(x_item, pooled, W1, b1, g1, be1, a1, W2, b2, g2, be2, a2, W3, b3):
    args = (x_item, pooled, W1, b1.reshape(1, -1), g1.reshape(1, -1),
            be1.reshape(1, -1), a1.reshape(1, -1), W2, b2.reshape(1, -1),
            g2.reshape(1, -1), be2.reshape(1, -1), a2.reshape(1, -1),
            W3, b3.reshape(1, -1))
    return pl.pallas_call(
        _mlp_body,
        out_shape=jax.ShapeDtypeStruct((B, OUT), jnp.float32),
    )(*args)


def kernel(i_goods_id, i_shop_id, i_cate_id, visited_goods_ids,
           visited_shop_ids, visited_cate_ids, emb_table, W1, b1, g1, be1, a1,
           W2, b2, g2, be2, a2, W3, b3):
    item_idx = jnp.stack([i_goods_id, i_shop_id, i_cate_id], axis=1)
    item_idx = item_idx.reshape(NW, IT_G, GROW)
    series_idx = jnp.stack(
        [visited_goods_ids, visited_shop_ids, visited_cate_ids], axis=2)
    series_idx = series_idx.reshape(NW, SE_NCH, SE_G, GROW)

    item_rows, series_rows = _sc_gather(item_idx, series_idx, emb_table)
    x_item = item_rows.reshape(B, 3 * E)
    x_series = series_rows.reshape(B, L, 3 * E)
    valid_mask = visited_goods_ids != 0

    pooled = _attention(valid_mask.astype(jnp.float32), x_item, x_series)
    output = _mlp(x_item, pooled, W1, b1, g1, be1, a1,
                  W2, b2, g2, be2, a2, W3, b3)
    return output, x_series, valid_mask


# speedup vs baseline: 2.8449x; 2.2019x over previous
import functools

import jax
import jax.numpy as jnp
from jax import lax
from jax.experimental import pallas as pl
from jax.experimental.pallas import tpu as pltpu
from jax.experimental.pallas import tpu_sc as plsc

B, L, E, V = 4096, 200, 16, 1000000
H1, H2, OUT = 200, 80, 2

NW = 32
IT_PER_W = 3 * B // NW
SE_CHUNK = 1536
SE_NCH = 50
SE_PER_W = SE_NCH * SE_CHUNK


POS = B * L
POS_PER_W = POS // NW
PCHUNK = 512
PNCH = POS_PER_W // PCHUNK
IB_PER_W = B // NW


def _gather_tec(ig_hbm, is_hbm, ic_hbm, vg_hbm, vs_hbm, vc_hbm, table_hbm,
                out_item, out_series, idx_v, rows_v, sem):
    wid = lax.axis_index("s") * 2 + lax.axis_index("c")

    it_base = wid * IB_PER_W
    for f, src in enumerate((ig_hbm, is_hbm, ic_hbm)):
        pltpu.sync_copy(src.at[pl.ds(it_base, IB_PER_W)],
                        idx_v.at[pl.ds(0, IB_PER_W)])
        pltpu.async_copy(table_hbm.at[idx_v.at[pl.ds(0, IB_PER_W)]],
                         rows_v.at[pl.ds(0, IB_PER_W)], sem).wait()
        pltpu.sync_copy(rows_v.at[pl.ds(0, IB_PER_W)],
                        out_item.at[pl.ds(it_base, IB_PER_W), f])

    def body(c, carry):
        off = wid * POS_PER_W + c * PCHUNK
        for f, src in enumerate((vg_hbm, vs_hbm, vc_hbm)):
            pltpu.sync_copy(src.at[pl.ds(off, PCHUNK)],
                            idx_v.at[pl.ds(0, PCHUNK)])
            pltpu.async_copy(table_hbm.at[idx_v.at[pl.ds(0, PCHUNK)]],
                             rows_v.at[pl.ds(0, PCHUNK)], sem).wait()
            pltpu.sync_copy(rows_v.at[pl.ds(0, PCHUNK)],
                            out_series.at[pl.ds(off, PCHUNK), f])
        return carry

    lax.fori_loop(0, PNCH, body, 0)


def _sc_gather(ig, i_s, ic, vg, vs, vc, table):
    mesh = plsc.VectorSubcoreMesh(core_axis_name="c", subcore_axis_name="s")
    f = functools.partial(
        pl.kernel, mesh=mesh,
        compiler_params=pltpu.CompilerParams(use_tc_tiling_on_sc=False),
        out_type=(
            jax.ShapeDtypeStruct((B, 3, E), jnp.float32),
            jax.ShapeDtypeStruct((POS, 3, E), jnp.float32),
        ),
        scratch_types=[
            pltpu.VMEM((PCHUNK,), jnp.int32),
            pltpu.VMEM((PCHUNK, E), jnp.float32),
            pltpu.SemaphoreType.DMA,
        ],
    )(_gather_tec)
    return f(ig, i_s, ic, vg, vs, vc, table)


def _attn_body(mf_ref, xi_ref, xs_ref, pooled_ref, xst_ref):
    j = pl.program_id(1)
    xs = xs_ref[...]
    xi = xi_ref[...]
    scores = jnp.sum(xs * xi[:, None, :], axis=2)
    ms = scores * mf_ref[...][:, :, 0]
    part = jnp.sum(xs * ms[:, :, None], axis=1)

    @pl.when(j == 0)
    def _init():
        pooled_ref[...] = part

    @pl.when(j != 0)
    def _acc():
        pooled_ref[...] += part

    xst_ref[...] = jnp.transpose(xs, (1, 2, 0))


def _attention(maskf, x_item, x_series, bb=256, lb=40):
    grid = (B // bb, L // lb)
    return pl.pallas_call(
        _attn_body,
        grid=grid,
        in_specs=[
            pl.BlockSpec((bb, lb, 1), lambda i, j: (i, j, 0)),
            pl.BlockSpec((bb, 3 * E), lambda i, j: (i, 0)),
            pl.BlockSpec((bb, lb, 3 * E), lambda i, j: (i, j, 0)),
        ],
        out_specs=[
            pl.BlockSpec((bb, 3 * E), lambda i, j: (i, 0)),
            pl.BlockSpec((lb, 3 * E, bb), lambda i, j: (j, 0, i)),
        ],
        out_shape=[
            jax.ShapeDtypeStruct((B, 3 * E), jnp.float32),
            jax.ShapeDtypeStruct((L, 3 * E, B), jnp.float32),
        ],
    )(maskf, x_item, x_series)


def _sigmoid(x):
    return 1.0 / (1.0 + jnp.exp(-x))


def _ln(x, gamma, beta, eps=1e-3):
    mu = jnp.mean(x, axis=-1, keepdims=True)
    var = jnp.mean((x - mu) ** 2, axis=-1, keepdims=True)
    return gamma * (x - mu) / jnp.sqrt(var + eps) + beta


def _dice_act(x, alpha, eps=1e-3):
    mu = jnp.mean(x, axis=0, keepdims=True)
    var = jnp.mean((x - mu) ** 2, axis=0, keepdims=True)
    xn = (x - mu) / jnp.sqrt(var + eps)
    p = _sigmoid(xn)
    return alpha * (1.0 - p) * x + p * x


def _mlp_body(xi_ref, pooled_ref, w1_ref, b1_ref, g1_ref, be1_ref, a1_ref,
              w2_ref, b2_ref, g2_ref, be2_ref, a2_ref, w3_ref, b3_ref, out_ref):
    x = jnp.concatenate([xi_ref[...], pooled_ref[...]], axis=1)
    h = jnp.dot(x, w1_ref[...], preferred_element_type=jnp.float32) + b1_ref[...]
    h = _ln(h, g1_ref[...], be1_ref[...])
    h = _dice_act(h, a1_ref[...])
    h = jnp.dot(h, w2_ref[...], preferred_element_type=jnp.float32) + b2_ref[...]
    h = _ln(h, g2_ref[...], be2_ref[...])
    h = _dice_act(h, a2_ref[...])
    logits = jnp.dot(h, w3_ref[...], preferred_element_type=jnp.float32) + b3_ref[...]
    m = jnp.max(logits, axis=-1, keepdims=True)
    e = jnp.exp(logits - m)
    out_ref[...] = e / jnp.sum(e, axis=-1, keepdims=True)


def _mlp(x_item, pooled, W1, b1, g1, be1, a1, W2, b2, g2, be2, a2, W3, b3):
    args = (x_item, pooled, W1, b1.reshape(1, -1), g1.reshape(1, -1),
            be1.reshape(1, -1), a1.reshape(1, -1), W2, b2.reshape(1, -1),
            g2.reshape(1, -1), be2.reshape(1, -1), a2.reshape(1, -1),
            W3, b3.reshape(1, -1))
    return pl.pallas_call(
        _mlp_body,
        out_shape=jax.ShapeDtypeStruct((B, OUT), jnp.float32),
    )(*args)


def kernel(i_goods_id, i_shop_id, i_cate_id, visited_goods_ids,
           visited_shop_ids, visited_cate_ids, emb_table, W1, b1, g1, be1, a1,
           W2, b2, g2, be2, a2, W3, b3):
    item_rows, series_rows = _sc_gather(
        i_goods_id, i_shop_id, i_cate_id,
        visited_goods_ids.reshape(-1), visited_shop_ids.reshape(-1),
        visited_cate_ids.reshape(-1), emb_table)
    x_item = item_rows.reshape(B, 3 * E)
    x_series = series_rows.reshape(B, L, 3 * E)
    valid_mask = visited_goods_ids != 0

    (pooled, xs_t) = _attention(
        valid_mask.astype(jnp.float32).reshape(B, L, 1), x_item, x_series)
    x_series_out = jnp.transpose(xs_t, (2, 0, 1))
    output = _mlp(x_item, pooled, W1, b1, g1, be1, a1,
                  W2, b2, g2, be2, a2, W3, b3)
    return output, x_series_out, valid_mask


# speedup vs baseline: 5.3974x; 1.8972x over previous
import functools

import jax
import jax.numpy as jnp
from jax import lax
from jax.experimental import pallas as pl
from jax.experimental.pallas import tpu as pltpu
from jax.experimental.pallas import tpu_sc as plsc

B, L, E, V = 4096, 200, 16, 1000000
H1, H2, OUT = 200, 80, 2

NW = 32
IT_PER_W = 3 * B // NW
SE_CHUNK = 1536
SE_NCH = 50
SE_PER_W = SE_NCH * SE_CHUNK


POS = B * L
POS_PER_W = POS // NW
PCHUNK = 800
PNCH = POS_PER_W // PCHUNK
IB_PER_W = B // NW


def _gather_tec(ig_hbm, is_hbm, ic_hbm, vg_hbm, vs_hbm, vc_hbm, table_hbm,
                out_item, out_series, idx_v, rows_v, sem):
    wid = lax.axis_index("s") * 2 + lax.axis_index("c")
    vsrc = (vg_hbm, vs_hbm, vc_hbm)

    it_base = wid * IB_PER_W
    for f, src in enumerate((ig_hbm, is_hbm, ic_hbm)):
        pltpu.sync_copy(src.at[pl.ds(it_base, IB_PER_W)],
                        idx_v.at[0, f, pl.ds(0, IB_PER_W)])
        pltpu.async_copy(table_hbm.at[idx_v.at[0, f, pl.ds(0, IB_PER_W)]],
                         rows_v.at[0, f, pl.ds(0, IB_PER_W)], sem).wait()
        pltpu.sync_copy(rows_v.at[0, f, pl.ds(0, IB_PER_W)],
                        out_item.at[pl.ds(it_base, IB_PER_W), f])

    def fire(c, s):
        off = wid * POS_PER_W + c * PCHUNK
        for f in range(3):
            pltpu.sync_copy(vsrc[f].at[pl.ds(off, PCHUNK)], idx_v.at[s, f])
            pltpu.async_copy(table_hbm.at[idx_v.at[s, f]],
                             rows_v.at[s, f], sem)

    def drain_write(c, s):
        off = wid * POS_PER_W + c * PCHUNK
        for f in range(3):
            pltpu.make_async_copy(table_hbm.at[idx_v.at[s, f]],
                                  rows_v.at[s, f], sem).wait()
            pltpu.sync_copy(rows_v.at[s, f],
                            out_series.at[pl.ds(off, PCHUNK), f])

    fire(0, 0)

    def body(i, carry):
        c0 = 2 * i
        fire(c0 + 1, 1)
        drain_write(c0, 0)

        @pl.when(c0 + 2 < PNCH)
        def _fire_next():
            fire(c0 + 2, 0)

        drain_write(c0 + 1, 1)
        return carry

    lax.fori_loop(0, PNCH // 2, body, 0)


def _sc_gather(ig, i_s, ic, vg, vs, vc, table):
    mesh = plsc.VectorSubcoreMesh(core_axis_name="c", subcore_axis_name="s")
    f = functools.partial(
        pl.kernel, mesh=mesh,
        compiler_params=pltpu.CompilerParams(use_tc_tiling_on_sc=False),
        out_type=(
            jax.ShapeDtypeStruct((B, 3, E), jnp.float32),
            jax.ShapeDtypeStruct((POS, 3, E), jnp.float32),
        ),
        scratch_types=[
            pltpu.VMEM((2, 3, PCHUNK), jnp.int32),
            pltpu.VMEM((2, 3, PCHUNK, E), jnp.float32),
            pltpu.SemaphoreType.DMA,
        ],
    )(_gather_tec)
    return f(ig, i_s, ic, vg, vs, vc, table)


def _attn_body(vgt_ref, xit_ref, xs_ref, pooled_ref, xst_ref):
    j = pl.program_id(1)
    xs2 = xs_ref[...]
    xst2 = jnp.transpose(xs2, (1, 0))
    xst = xst2.reshape(-1, 3 * E, xst2.shape[-1])
    xst_ref[...] = xst
    xi_t = xit_ref[...]
    scores_t = jnp.sum(xst * xi_t[None, :, :], axis=1)
    maskf_t = (vgt_ref[...] != 0).astype(jnp.float32)
    ms_t = scores_t * maskf_t
    part_t = jnp.sum(xst * ms_t[:, None, :], axis=0)

    @pl.when(j == 0)
    def _init():
        pooled_ref[...] = part_t

    @pl.when(j != 0)
    def _acc():
        pooled_ref[...] += part_t


def _attention(vg_t, x_item_t, x_series_flat, bb=256, lb=40):
    grid = (B // bb, L // lb)
    return pl.pallas_call(
        _attn_body,
        grid=grid,
        in_specs=[
            pl.BlockSpec((lb, bb), lambda i, j: (j, i)),
            pl.BlockSpec((3 * E, bb), lambda i, j: (0, i)),
            pl.BlockSpec((bb, lb * 3 * E), lambda i, j: (i, j)),
        ],
        out_specs=[
            pl.BlockSpec((3 * E, bb), lambda i, j: (0, i)),
            pl.BlockSpec((lb, 3 * E, bb), lambda i, j: (j, 0, i)),
        ],
        out_shape=[
            jax.ShapeDtypeStruct((3 * E, B), jnp.float32),
            jax.ShapeDtypeStruct((L, 3 * E, B), jnp.float32),
        ],
    )(vg_t, x_item_t, x_series_flat)


def _sigmoid(x):
    return 1.0 / (1.0 + jnp.exp(-x))


def _ln(x, gamma, beta, eps=1e-3):
    mu = jnp.mean(x, axis=-1, keepdims=True)
    var = jnp.mean((x - mu) ** 2, axis=-1, keepdims=True)
    return gamma * (x - mu) / jnp.sqrt(var + eps) + beta


def _dice_act(x, alpha, eps=1e-3):
    mu = jnp.mean(x, axis=0, keepdims=True)
    var = jnp.mean((x - mu) ** 2, axis=0, keepdims=True)
    xn = (x - mu) / jnp.sqrt(var + eps)
    p = _sigmoid(xn)
    return alpha * (1.0 - p) * x + p * x


def _mlp_body(xi_ref, pooled_ref, w1a_ref, w1b_ref, b1_ref, g1_ref, be1_ref,
              a1_ref, w2_ref, b2_ref, g2_ref, be2_ref, a2_ref, w3_ref, b3_ref,
              out_ref):
    h = (jnp.dot(xi_ref[...], w1a_ref[...],
                 preferred_element_type=jnp.float32)
         + lax.dot_general(pooled_ref[...], w1b_ref[...],
                           (((0,), (0,)), ((), ())),
                           preferred_element_type=jnp.float32)
         + b1_ref[...])
    h = _ln(h, g1_ref[...], be1_ref[...])
    h = _dice_act(h, a1_ref[...])
    h = jnp.dot(h, w2_ref[...], preferred_element_type=jnp.float32) + b2_ref[...]
    h = _ln(h, g2_ref[...], be2_ref[...])
    h = _dice_act(h, a2_ref[...])
    logits = jnp.dot(h, w3_ref[...], preferred_element_type=jnp.float32) + b3_ref[...]
    m = jnp.max(logits, axis=-1, keepdims=True)
    e = jnp.exp(logits - m)
    out_ref[...] = e / jnp.sum(e, axis=-1, keepdims=True)


def _mlp(x_item, pooled_t, W1, b1, g1, be1, a1, W2, b2, g2, be2, a2, W3, b3):
    args = (x_item, pooled_t, W1[:3 * E], W1[3 * E:], b1.reshape(1, -1),
            g1.reshape(1, -1),
            be1.reshape(1, -1), a1.reshape(1, -1), W2, b2.reshape(1, -1),
            g2.reshape(1, -1), be2.reshape(1, -1), a2.reshape(1, -1),
            W3, b3.reshape(1, -1))
    return pl.pallas_call(
        _mlp_body,
        out_shape=jax.ShapeDtypeStruct((B, OUT), jnp.float32),
    )(*args)


def kernel(i_goods_id, i_shop_id, i_cate_id, visited_goods_ids,
           visited_shop_ids, visited_cate_ids, emb_table, W1, b1, g1, be1, a1,
           W2, b2, g2, be2, a2, W3, b3):
    item_rows, series_rows = _sc_gather(
        i_goods_id, i_shop_id, i_cate_id,
        visited_goods_ids.reshape(-1), visited_shop_ids.reshape(-1),
        visited_cate_ids.reshape(-1), emb_table)
    x_item = item_rows.reshape(B, 3 * E)
    x_series_flat = series_rows.reshape(B, L * 3 * E)
    valid_mask = visited_goods_ids != 0

    (pooled_t, xs_t) = _attention(
        jnp.transpose(visited_goods_ids), jnp.transpose(x_item),
        x_series_flat)
    x_series_out = jnp.transpose(xs_t, (2, 0, 1))
    output = _mlp(x_item, pooled_t, W1, b1, g1, be1, a1,
                  W2, b2, g2, be2, a2, W3, b3)
    return output, x_series_out, valid_mask


# speedup vs baseline: 5.5246x; 1.0236x over previous
import functools

import jax
import jax.numpy as jnp
from jax import lax
from jax.experimental import pallas as pl
from jax.experimental.pallas import tpu as pltpu
from jax.experimental.pallas import tpu_sc as plsc

B, L, E, V = 4096, 200, 16, 1000000
H1, H2, OUT = 200, 80, 2

NW = 32
IT_PER_W = 3 * B // NW
SE_CHUNK = 1536
SE_NCH = 50
SE_PER_W = SE_NCH * SE_CHUNK


POS = B * L
POS_PER_W = POS // NW
PCHUNK = 800
PNCH = POS_PER_W // PCHUNK
IB_PER_W = B // NW


def _gather_tec(ig_hbm, is_hbm, ic_hbm, vg_hbm, vs_hbm, vc_hbm, table_hbm,
                out_item, out_series, idx_v, rows_v, sem):
    wid = lax.axis_index("s") * 2 + lax.axis_index("c")
    vsrc = (vg_hbm, vs_hbm, vc_hbm)

    it_base = wid * IB_PER_W
    for f, src in enumerate((ig_hbm, is_hbm, ic_hbm)):
        pltpu.sync_copy(src.at[pl.ds(it_base, IB_PER_W)],
                        idx_v.at[0, f, pl.ds(0, IB_PER_W)])
        pltpu.async_copy(table_hbm.at[idx_v.at[0, f, pl.ds(0, IB_PER_W)]],
                         rows_v.at[0, f, pl.ds(0, IB_PER_W)], sem).wait()
        pltpu.sync_copy(rows_v.at[0, f, pl.ds(0, IB_PER_W)],
                        out_item.at[pl.ds(it_base, IB_PER_W), f])

    def fire(c, s):
        off = wid * POS_PER_W + c * PCHUNK
        for f in range(3):
            pltpu.sync_copy(vsrc[f].at[pl.ds(off, PCHUNK)], idx_v.at[s, f])
            pltpu.async_copy(table_hbm.at[idx_v.at[s, f]],
                             rows_v.at[s, f], sem)

    def drain_write(c, s):
        off = wid * POS_PER_W + c * PCHUNK
        for f in range(3):
            pltpu.make_async_copy(table_hbm.at[idx_v.at[s, f]],
                                  rows_v.at[s, f], sem).wait()
            pltpu.sync_copy(rows_v.at[s, f],
                            out_series.at[pl.ds(off, PCHUNK), f])

    fire(0, 0)

    def body(i, carry):
        c0 = 2 * i
        fire(c0 + 1, 1)
        drain_write(c0, 0)

        @pl.when(c0 + 2 < PNCH)
        def _fire_next():
            fire(c0 + 2, 0)

        drain_write(c0 + 1, 1)
        return carry

    lax.fori_loop(0, PNCH // 2, body, 0)


def _sc_gather(ig, i_s, ic, vg, vs, vc, table):
    mesh = plsc.VectorSubcoreMesh(core_axis_name="c", subcore_axis_name="s")
    f = functools.partial(
        pl.kernel, mesh=mesh,
        compiler_params=pltpu.CompilerParams(use_tc_tiling_on_sc=False),
        out_type=(
            jax.ShapeDtypeStruct((B, 3, E), jnp.float32),
            jax.ShapeDtypeStruct((POS, 3, E), jnp.float32),
        ),
        scratch_types=[
            pltpu.VMEM((2, 3, PCHUNK), jnp.int32),
            pltpu.VMEM((2, 3, PCHUNK, E), jnp.float32),
            pltpu.SemaphoreType.DMA,
        ],
    )(_gather_tec)
    return f(ig, i_s, ic, vg, vs, vc, table)


def _attn_body(vgt_ref, xit_ref, xs_ref, pooled_ref, xst_ref):
    j = pl.program_id(1)
    xs2 = xs_ref[...]
    xst2 = jnp.transpose(xs2, (1, 0))
    xst = xst2.reshape(-1, 3 * E, xst2.shape[-1])
    xst_ref[...] = xst
    xi_t = xit_ref[...]
    scores_t = jnp.sum(xst * xi_t[None, :, :], axis=1)
    maskf_t = (vgt_ref[...] != 0).astype(jnp.float32)
    ms_t = scores_t * maskf_t
    part_t = jnp.sum(xst * ms_t[:, None, :], axis=0)

    @pl.when(j == 0)
    def _init():
        pooled_ref[...] = part_t

    @pl.when(j != 0)
    def _acc():
        pooled_ref[...] += part_t


def _attention(vg_t, x_item_t, x_series_flat, bb=512, lb=40):
    grid = (B // bb, L // lb)
    return pl.pallas_call(
        _attn_body,
        grid=grid,
        in_specs=[
            pl.BlockSpec((lb, bb), lambda i, j: (j, i)),
            pl.BlockSpec((3 * E, bb), lambda i, j: (0, i)),
            pl.BlockSpec((bb, lb * 3 * E), lambda i, j: (i, j)),
        ],
        out_specs=[
            pl.BlockSpec((3 * E, bb), lambda i, j: (0, i)),
            pl.BlockSpec((lb, 3 * E, bb), lambda i, j: (j, 0, i)),
        ],
        out_shape=[
            jax.ShapeDtypeStruct((3 * E, B), jnp.float32),
            jax.ShapeDtypeStruct((L, 3 * E, B), jnp.float32),
        ],
    )(vg_t, x_item_t, x_series_flat)


def _sigmoid(x):
    return 1.0 / (1.0 + jnp.exp(-x))


def _ln(x, gamma, beta, eps=1e-3):
    mu = jnp.mean(x, axis=-1, keepdims=True)
    var = jnp.mean((x - mu) ** 2, axis=-1, keepdims=True)
    return gamma * (x - mu) / jnp.sqrt(var + eps) + beta


def _dice_act(x, alpha, eps=1e-3):
    mu = jnp.mean(x, axis=0, keepdims=True)
    var = jnp.mean((x - mu) ** 2, axis=0, keepdims=True)
    xn = (x - mu) / jnp.sqrt(var + eps)
    p = _sigmoid(xn)
    return alpha * (1.0 - p) * x + p * x


def _mlp_body(xi_ref, pooled_ref, w1a_ref, w1b_ref, b1_ref, g1_ref, be1_ref,
              a1_ref, w2_ref, b2_ref, g2_ref, be2_ref, a2_ref, w3_ref, b3_ref,
              out_ref):
    h = (jnp.dot(xi_ref[...], w1a_ref[...],
                 preferred_element_type=jnp.float32)
         + lax.dot_general(pooled_ref[...], w1b_ref[...],
                           (((0,), (0,)), ((), ())),
                           preferred_element_type=jnp.float32)
         + b1_ref[...])
    h = _ln(h, g1_ref[...], be1_ref[...])
    h = _dice_act(h, a1_ref[...])
    h = jnp.dot(h, w2_ref[...], preferred_element_type=jnp.float32) + b2_ref[...]
    h = _ln(h, g2_ref[...], be2_ref[...])
    h = _dice_act(h, a2_ref[...])
    logits = jnp.dot(h, w3_ref[...], preferred_element_type=jnp.float32) + b3_ref[...]
    m = jnp.max(logits, axis=-1, keepdims=True)
    e = jnp.exp(logits - m)
    out_ref[...] = e / jnp.sum(e, axis=-1, keepdims=True)


def _mlp(x_item, pooled_t, W1, b1, g1, be1, a1, W2, b2, g2, be2, a2, W3, b3):
    args = (x_item, pooled_t, W1[:3 * E], W1[3 * E:], b1.reshape(1, -1),
            g1.reshape(1, -1),
            be1.reshape(1, -1), a1.reshape(1, -1), W2, b2.reshape(1, -1),
            g2.reshape(1, -1), be2.reshape(1, -1), a2.reshape(1, -1),
            W3, b3.reshape(1, -1))
    return pl.pallas_call(
        _mlp_body,
        out_shape=jax.ShapeDtypeStruct((B, OUT), jnp.float32),
    )(*args)


def kernel(i_goods_id, i_shop_id, i_cate_id, visited_goods_ids,
           visited_shop_ids, visited_cate_ids, emb_table, W1, b1, g1, be1, a1,
           W2, b2, g2, be2, a2, W3, b3):
    item_rows, series_rows = _sc_gather(
        i_goods_id, i_shop_id, i_cate_id,
        visited_goods_ids.reshape(-1), visited_shop_ids.reshape(-1),
        visited_cate_ids.reshape(-1), emb_table)
    x_item = item_rows.reshape(B, 3 * E)
    x_series_flat = series_rows.reshape(B, L * 3 * E)
    valid_mask = visited_goods_ids != 0

    (pooled_t, xs_t) = _attention(
        jnp.transpose(visited_goods_ids), jnp.transpose(x_item),
        x_series_flat)
    x_series_out = jnp.transpose(xs_t, (2, 0, 1))
    output = _mlp(x_item, pooled_t, W1, b1, g1, be1, a1,
                  W2, b2, g2, be2, a2, W3, b3)
    return output, x_series_out, valid_mask
